# baseline (speedup 1.0000x reference)
import functools
import math

import jax
import jax.numpy as jnp
from jax.experimental import pallas as pl
from jax.experimental.pallas import tpu as pltpu

_LANES = 128
_SUB = 8


def _mse_body(o_ref, l_ref, res_ref, acc_ref, val_ref, sem_ref, *, inv_n,
              acc_rows):
    i = pl.program_id(0)

    @pl.when(i == 0)
    def _():
        acc_ref[...] = jnp.zeros_like(acc_ref)

    d = o_ref[...] - l_ref[...]
    sq = d * d
    acc_ref[...] += jnp.sum(sq.reshape(-1, acc_rows, _LANES), axis=0)

    @pl.when(i == pl.num_programs(0) - 1)
    def _():
        val_ref[...] = jnp.sum(acc_ref[...], keepdims=True) * inv_n
        cp = pltpu.make_async_copy(val_ref, res_ref, sem_ref)
        cp.start()
        cp.wait()


def kernel(outputs, labels):
    assert outputs.shape == labels.shape
    out_dtype = outputs.dtype
    n_total = math.prod(outputs.shape) if outputs.shape else 1

    rows = -(-n_total // _LANES)
    rows = -(-rows // _SUB) * _SUB

    tile_rows = min(16384, rows)
    steps = -(-rows // tile_rows)
    rows_padded = steps * tile_rows
    acc_rows = 256 if tile_rows % 256 == 0 else _SUB

    def to2d(x):
        flat = x.astype(jnp.float32).reshape(-1)
        pad = rows_padded * _LANES - flat.shape[0]
        if pad:
            flat = jnp.pad(flat, (0, pad))
        return flat.reshape(rows_padded, _LANES)

    o2d = to2d(outputs)
    l2d = to2d(labels)

    inv_n = 1.0 / float(n_total)
    body = functools.partial(_mse_body, inv_n=inv_n, acc_rows=acc_rows)

    res = pl.pallas_call(
        body,
        out_shape=jax.ShapeDtypeStruct((1, 1), jnp.float32),
        grid_spec=pltpu.PrefetchScalarGridSpec(
            num_scalar_prefetch=0,
            grid=(steps,),
            in_specs=[
                pl.BlockSpec((tile_rows, _LANES), lambda i: (i, 0)),
                pl.BlockSpec((tile_rows, _LANES), lambda i: (i, 0)),
            ],
            out_specs=pl.BlockSpec(memory_space=pl.ANY),
            scratch_shapes=[
                pltpu.VMEM((acc_rows, _LANES), jnp.float32),
                pltpu.VMEM((1, 1), jnp.float32),
                pltpu.SemaphoreType.DMA,
            ],
        ),
        compiler_params=pltpu.CompilerParams(
            dimension_semantics=("arbitrary",)
        ),
    )(o2d, l2d)

    return res.reshape(()).astype(out_dtype)


# speedup vs baseline: 1.0315x; 1.0315x over previous
import functools
import math

import jax
import jax.numpy as jnp
from jax.experimental import pallas as pl
from jax.experimental.pallas import tpu as pltpu

_LANES = 128
_SUB = 8


def _mse_body(o_ref, l_ref, res_ref, acc_ref, val_ref, sem_ref, *, inv_n,
              acc_rows):
    i = pl.program_id(0)

    @pl.when(i == 0)
    def _():
        acc_ref[...] = jnp.zeros_like(acc_ref)

    d = o_ref[...] - l_ref[...]
    sq = d * d
    acc_ref[...] += jnp.sum(sq.reshape(-1, acc_rows, _LANES), axis=0)

    @pl.when(i == pl.num_programs(0) - 1)
    def _():
        val_ref[...] = jnp.sum(acc_ref[...], keepdims=True) * inv_n
        cp = pltpu.make_async_copy(val_ref, res_ref, sem_ref)
        cp.start()
        cp.wait()


def kernel(outputs, labels):
    assert outputs.shape == labels.shape
    out_dtype = outputs.dtype
    n_total = math.prod(outputs.shape) if outputs.shape else 1

    rows = -(-n_total // _LANES)
    rows = -(-rows // _SUB) * _SUB

    tile_rows = min(8192, rows)
    steps = -(-rows // tile_rows)
    rows_padded = steps * tile_rows
    acc_rows = 256 if tile_rows % 256 == 0 else _SUB

    def to2d(x):
        flat = x.astype(jnp.float32).reshape(-1)
        pad = rows_padded * _LANES - flat.shape[0]
        if pad:
            flat = jnp.pad(flat, (0, pad))
        return flat.reshape(rows_padded, _LANES)

    o2d = to2d(outputs)
    l2d = to2d(labels)

    inv_n = 1.0 / float(n_total)
    body = functools.partial(_mse_body, inv_n=inv_n, acc_rows=acc_rows)

    res = pl.pallas_call(
        body,
        out_shape=jax.ShapeDtypeStruct((1, 1), jnp.float32),
        grid_spec=pltpu.PrefetchScalarGridSpec(
            num_scalar_prefetch=0,
            grid=(steps,),
            in_specs=[
                pl.BlockSpec((tile_rows, _LANES), lambda i: (i, 0)),
                pl.BlockSpec((tile_rows, _LANES), lambda i: (i, 0)),
            ],
            out_specs=pl.BlockSpec(memory_space=pl.ANY),
            scratch_shapes=[
                pltpu.VMEM((acc_rows, _LANES), jnp.float32),
                pltpu.VMEM((1, 1), jnp.float32),
                pltpu.SemaphoreType.DMA,
            ],
        ),
        compiler_params=pltpu.CompilerParams(
            dimension_semantics=("arbitrary",)
        ),
    )(o2d, l2d)

    return res.reshape(()).astype(out_dtype)
